# baseline (speedup 1.0000x reference)
import functools

import jax
import jax.numpy as jnp
from jax import lax
from jax.experimental import pallas as pl
from jax.experimental.pallas import tpu as pltpu
from jax.experimental.pallas import tpu_sc as plsc

N = 10000
NP = 10240
E = 320000
D = 128
H = 32
HH = H * H
LANES = 128
K = 80
NW = 32
EP = NW * K * LANES
NSUB = 16
RPT = NP // NSUB
TPT = N // NSUB
DEGW = 16
BR = 1000
GRID = N // BR

f32 = jnp.float32
i32 = jnp.int32

_MESH = plsc.VectorSubcoreMesh(core_axis_name="c", subcore_axis_name="s")
_SC_PARAMS = pltpu.CompilerParams(use_tc_tiling_on_sc=False)


def _sc_deg_body(col_hbm, ones_hbm, z_hbm, out_hbm, acc, colv, ones_v):
    c = lax.axis_index("c")
    s = lax.axis_index("s")
    w = c * NSUB + s
    tb = s * RPT
    pltpu.sync_copy(col_hbm.at[pl.ds(w * K, K)], colv)
    pltpu.sync_copy(ones_hbm, ones_v)
    pltpu.sync_copy(z_hbm, acc.at[pl.ds(tb, RPT)])
    plsc.subcore_barrier()

    def step(k, carry):
        pltpu.sync_copy(ones_v, acc.at[colv.at[k]], add=True)
        return carry

    lax.fori_loop(0, K, step, 0)
    plsc.subcore_barrier()
    pltpu.sync_copy(acc.at[pl.ds(tb, RPT)], out_hbm.at[c, pl.ds(tb, RPT)])


_sc_deg = functools.partial(
    pl.kernel,
    out_type=jax.ShapeDtypeStruct((2, NP, DEGW), f32),
    mesh=_MESH,
    scratch_types=[
        pltpu.VMEM_SHARED((NP, DEGW), f32),
        pltpu.VMEM((K, LANES), i32),
        pltpu.VMEM((LANES, DEGW), f32),
    ],
    compiler_params=_SC_PARAMS,
)(_sc_deg_body)


NB = 4
NG = K // NB


def _sc_prop_body(tab_hbm, row_hbm, col_hbm, z_hbm, out_hbm,
                  acc, tab_sh, rowv, colv,
                  buf0, buf1, buf2, buf3,
                  sg0, sg1, sg2, sg3, ss0, ss1, ss2, ss3):
    c = lax.axis_index("c")
    s = lax.axis_index("s")
    w = c * NSUB + s
    tb = s * RPT
    bufs = (buf0, buf1, buf2, buf3)
    sg = (sg0, sg1, sg2, sg3)
    ss = (ss0, ss1, ss2, ss3)
    h1 = pltpu.async_copy(row_hbm.at[pl.ds(w * K, K)], rowv, sg0)
    h2 = pltpu.async_copy(col_hbm.at[pl.ds(w * K, K)], colv, sg1)
    h3 = pltpu.async_copy(z_hbm, acc.at[pl.ds(tb, RPT)], sg2)
    h4 = pltpu.async_copy(tab_hbm.at[pl.ds(s * TPT, TPT)],
                          tab_sh.at[pl.ds(s * TPT, TPT)], sg3)
    h1.wait()
    h2.wait()
    h3.wait()
    h4.wait()
    plsc.subcore_barrier()

    for b in range(NB):
        pltpu.async_copy(tab_sh.at[rowv.at[b]], bufs[b], sg[b])

    def group(g, carry):
        for b in range(NB):
            j = g * NB + b
            pltpu.make_async_copy(tab_sh.at[rowv.at[j]], bufs[b],
                                  sg[b]).wait()
            pltpu.async_copy(bufs[b], acc.at[colv.at[j]], ss[b], add=True)
        for b in range(NB):
            j = g * NB + b
            pltpu.make_async_copy(bufs[b], acc.at[colv.at[j]],
                                  ss[b]).wait()
            pltpu.async_copy(tab_sh.at[rowv.at[j + NB]], bufs[b], sg[b])
        return carry

    lax.fori_loop(0, NG - 1, group, 0)
    for b in range(NB):
        j = (NG - 1) * NB + b
        pltpu.make_async_copy(tab_sh.at[rowv.at[j]], bufs[b], sg[b]).wait()
        pltpu.async_copy(bufs[b], acc.at[colv.at[j]], ss[b], add=True)
    for b in range(NB):
        j = (NG - 1) * NB + b
        pltpu.make_async_copy(bufs[b], acc.at[colv.at[j]], ss[b]).wait()
    plsc.subcore_barrier()
    pltpu.sync_copy(acc.at[pl.ds(tb, RPT)], out_hbm.at[c, pl.ds(tb, RPT)])


_sc_prop = functools.partial(
    pl.kernel,
    out_type=jax.ShapeDtypeStruct((2, NP, H), f32),
    mesh=_MESH,
    scratch_types=[
        pltpu.VMEM_SHARED((NP, H), f32),
        pltpu.VMEM_SHARED((N, H), f32),
        pltpu.VMEM((K, LANES), i32),
        pltpu.VMEM((K, LANES), i32),
        pltpu.VMEM((LANES, H), f32),
        pltpu.VMEM((LANES, H), f32),
        pltpu.VMEM((LANES, H), f32),
        pltpu.VMEM((LANES, H), f32),
        pltpu.SemaphoreType.DMA,
        pltpu.SemaphoreType.DMA,
        pltpu.SemaphoreType.DMA,
        pltpu.SemaphoreType.DMA,
        pltpu.SemaphoreType.DMA,
        pltpu.SemaphoreType.DMA,
        pltpu.SemaphoreType.DMA,
        pltpu.SemaphoreType.DMA,
    ],
    compiler_params=_SC_PARAMS,
)(_sc_prop_body)


def _dis_from(degp_ref):
    deg = degp_ref[0, :, 0:1] + degp_ref[1, :, 0:1] + 1.0
    return lax.rsqrt(deg)


def _dot(a, b):
    return jnp.dot(a, b, preferred_element_type=f32,
                   precision=lax.Precision.DEFAULT)


def _tc1_body(x_ref, degp_ref, w0_ref, o_ref):
    dis = _dis_from(degp_ref)
    o_ref[...] = _dot(x_ref[...], w0_ref[...]) * dis


def _tc2_body(s_ref, hp_ref, degp_ref, bg_ref, w1_ref, b1_ref, w2_ref,
              b2_ref, wg_ref, o_ref):
    dis = _dis_from(degp_ref)
    g = (s_ref[0] + s_ref[1] + hp_ref[...]) * dis + bg_ref[...]
    z = jnp.maximum(_dot(g, w1_ref[...]) + b1_ref[...], 0.0)
    hr = jnp.maximum(_dot(z, w2_ref[...]) + b2_ref[...], 0.0)
    o_ref[...] = _dot(hr, wg_ref[...]) * dis


def _tc3_body(s_ref, hp_ref, degp_ref, bg_ref, w1_ref, b1_ref, w2_ref,
              b2_ref, o_ref):
    dis = _dis_from(degp_ref)
    g = (s_ref[0] + s_ref[1] + hp_ref[...]) * dis + bg_ref[...]
    z = jnp.maximum(_dot(g, w1_ref[...]) + b1_ref[...], 0.0)
    o_ref[...] = _dot(z, w2_ref[...]) + b2_ref[...]


def _full(shape):
    return pl.BlockSpec(shape, lambda i: tuple(0 for _ in shape))


def _tc1(x_p, degp, W0):
    return pl.pallas_call(
        _tc1_body,
        grid=(GRID,),
        in_specs=[
            pl.BlockSpec((BR, D), lambda i: (i, 0)),
            pl.BlockSpec((2, BR, DEGW), lambda i: (0, i, 0)),
            _full((D, H)),
        ],
        out_specs=pl.BlockSpec((BR, H), lambda i: (i, 0)),
        out_shape=jax.ShapeDtypeStruct((NP, H), f32),
    )(x_p, degp, W0)


def _tc2(s0, h0p, degp, bg, W1, b1, W2, b2, Wg):
    return pl.pallas_call(
        _tc2_body,
        grid=(GRID,),
        in_specs=[
            pl.BlockSpec((2, BR, H), lambda i: (0, i, 0)),
            pl.BlockSpec((BR, H), lambda i: (i, 0)),
            pl.BlockSpec((2, BR, DEGW), lambda i: (0, i, 0)),
            _full((1, H)),
            _full((H, H)),
            _full((1, H)),
            _full((H, HH)),
            _full((1, HH)),
            _full((HH, H)),
        ],
        out_specs=pl.BlockSpec((BR, H), lambda i: (i, 0)),
        out_shape=jax.ShapeDtypeStruct((NP, H), f32),
    )(s0, h0p, degp, bg, W1, b1, W2, b2, Wg)


def _tc3(s1, h1p, degp, bg, W1, b1, W2, b2):
    return pl.pallas_call(
        _tc3_body,
        grid=(GRID,),
        in_specs=[
            pl.BlockSpec((2, BR, H), lambda i: (0, i, 0)),
            pl.BlockSpec((BR, H), lambda i: (i, 0)),
            pl.BlockSpec((2, BR, DEGW), lambda i: (0, i, 0)),
            _full((1, H)),
            _full((H, H)),
            _full((1, H)),
            _full((H, HH)),
            _full((1, HH)),
        ],
        out_specs=pl.BlockSpec((BR, HH), lambda i: (i, 0)),
        out_shape=jax.ShapeDtypeStruct((NP, HH), f32),
    )(s1, h1p, degp, bg, W1, b1, W2, b2)


def kernel(x, edge_index,
           W_gcn0, b_gcn0, W_mlp0_1, b_mlp0_1, W_mlp0_2, b_mlp0_2,
           W_gcn1, b_gcn1, W_mlp1_1, b_mlp1_1, W_mlp1_2, b_mlp1_2):
    row = edge_index[0].astype(i32)
    col = edge_index[1].astype(i32)
    pad = EP - E
    row_p = jnp.concatenate([row, jnp.zeros((pad,), i32)])
    col_p = jnp.concatenate(
        [col, N + (jnp.arange(pad, dtype=i32) % (NP - N))])
    row_p = row_p.reshape(EP // LANES, LANES)
    col_p = col_p.reshape(EP // LANES, LANES)
    x_p = jnp.pad(x, ((0, NP - N), (0, 0)))
    z_deg = jnp.zeros((RPT, DEGW), f32)
    z_h = jnp.zeros((RPT, H), f32)
    ones_deg = jnp.ones((LANES, DEGW), f32)

    degp = _sc_deg(col_p, ones_deg, z_deg)
    h0p = _tc1(x_p, degp, W_gcn0)
    s0 = _sc_prop(h0p, row_p, col_p, z_h)
    h1p = _tc2(s0, h0p, degp, b_gcn0.reshape(1, H),
               W_mlp0_1, b_mlp0_1.reshape(1, H),
               W_mlp0_2, b_mlp0_2.reshape(1, HH), W_gcn1)
    s1 = _sc_prop(h1p, row_p, col_p, z_h)
    out = _tc3(s1, h1p, degp, b_gcn1.reshape(1, H),
               W_mlp1_1, b_mlp1_1.reshape(1, H),
               W_mlp1_2, b_mlp1_2.reshape(1, HH))
    return out[:N]


# speedup vs baseline: 1.0711x; 1.0711x over previous
import functools

import jax
import jax.numpy as jnp
from jax import lax
from jax.experimental import pallas as pl
from jax.experimental.pallas import tpu as pltpu
from jax.experimental.pallas import tpu_sc as plsc

N = 10000
NP = 10240
E = 320000
D = 128
H = 32
HH = H * H
LANES = 128
K = 80
NW = 32
EP = NW * K * LANES
NSUB = 16
RPT = NP // NSUB
TPT = N // NSUB
DEGW = 16
BR = 1000
GRID = N // BR

f32 = jnp.float32
i32 = jnp.int32

_MESH = plsc.VectorSubcoreMesh(core_axis_name="c", subcore_axis_name="s")
_SC_PARAMS = pltpu.CompilerParams(use_tc_tiling_on_sc=False)


def _sc_deg_body(col_hbm, ones_hbm, z_hbm, out_hbm, acc, colv, ones_v):
    c = lax.axis_index("c")
    s = lax.axis_index("s")
    w = c * NSUB + s
    tb = s * RPT
    pltpu.sync_copy(col_hbm.at[pl.ds(w * K, K)], colv)
    pltpu.sync_copy(ones_hbm, ones_v)
    pltpu.sync_copy(z_hbm, acc.at[pl.ds(tb, RPT)])
    plsc.subcore_barrier()

    def step(k, carry):
        pltpu.sync_copy(ones_v, acc.at[colv.at[k]], add=True)
        return carry

    lax.fori_loop(0, K, step, 0)
    plsc.subcore_barrier()
    pltpu.sync_copy(acc.at[pl.ds(tb, RPT)], out_hbm.at[c, pl.ds(tb, RPT)])


_sc_deg = functools.partial(
    pl.kernel,
    out_type=jax.ShapeDtypeStruct((2, NP, DEGW), f32),
    mesh=_MESH,
    scratch_types=[
        pltpu.VMEM_SHARED((NP, DEGW), f32),
        pltpu.VMEM((K, LANES), i32),
        pltpu.VMEM((LANES, DEGW), f32),
    ],
    compiler_params=_SC_PARAMS,
)(_sc_deg_body)


def _sc_prop_body(tab_hbm, row_hbm, col_hbm, z_hbm, out_hbm,
                  acc, tab_sh, rowv, colv, buf0, buf1,
                  sg0, sg1, sg2, sg3):
    c = lax.axis_index("c")
    s = lax.axis_index("s")
    w = c * NSUB + s
    tb = s * RPT
    h1 = pltpu.async_copy(row_hbm.at[pl.ds(w * K, K)], rowv, sg0)
    h2 = pltpu.async_copy(col_hbm.at[pl.ds(w * K, K)], colv, sg1)
    h3 = pltpu.async_copy(z_hbm, acc.at[pl.ds(tb, RPT)], sg2)
    h4 = pltpu.async_copy(tab_hbm.at[pl.ds(s * TPT, TPT)],
                          tab_sh.at[pl.ds(s * TPT, TPT)], sg3)
    h1.wait()
    h2.wait()
    h3.wait()
    h4.wait()
    plsc.subcore_barrier()

    pltpu.async_copy(tab_sh.at[rowv.at[0]], buf0, sg0)

    def pair(k, carry):
        j0 = 2 * k
        j1 = j0 + 1
        pltpu.async_copy(tab_sh.at[rowv.at[j1]], buf1, sg1)
        pltpu.make_async_copy(tab_sh.at[rowv.at[j0]], buf0, sg0).wait()
        pltpu.sync_copy(buf0, acc.at[colv.at[j0]], add=True)

        @pl.when(j1 + 1 < K)
        def _():
            pltpu.async_copy(tab_sh.at[rowv.at[j1 + 1]], buf0, sg0)

        pltpu.make_async_copy(tab_sh.at[rowv.at[j1]], buf1, sg1).wait()
        pltpu.sync_copy(buf1, acc.at[colv.at[j1]], add=True)
        return carry

    lax.fori_loop(0, K // 2, pair, 0)
    plsc.subcore_barrier()
    pltpu.sync_copy(acc.at[pl.ds(tb, RPT)], out_hbm.at[c, pl.ds(tb, RPT)])


_sc_prop = functools.partial(
    pl.kernel,
    out_type=jax.ShapeDtypeStruct((2, NP, H), f32),
    mesh=_MESH,
    scratch_types=[
        pltpu.VMEM_SHARED((NP, H), f32),
        pltpu.VMEM_SHARED((N, H), f32),
        pltpu.VMEM((K, LANES), i32),
        pltpu.VMEM((K, LANES), i32),
        pltpu.VMEM((LANES, H), f32),
        pltpu.VMEM((LANES, H), f32),
        pltpu.SemaphoreType.DMA,
        pltpu.SemaphoreType.DMA,
        pltpu.SemaphoreType.DMA,
        pltpu.SemaphoreType.DMA,
    ],
    compiler_params=_SC_PARAMS,
)(_sc_prop_body)


def _dis_from(degp_ref):
    deg = degp_ref[0, :, 0:1] + degp_ref[1, :, 0:1] + 1.0
    return lax.rsqrt(deg)


def _dot(a, b):
    return jnp.dot(a, b, preferred_element_type=f32,
                   precision=lax.Precision.DEFAULT)


def _tc_mm_body(x_ref, w0_ref, o_ref):
    o_ref[...] = _dot(x_ref[...], w0_ref[...])


def _tc_scale_body(u_ref, degp_ref, o_ref):
    o_ref[...] = u_ref[...] * _dis_from(degp_ref)


def _tc2_body(s_ref, hp_ref, degp_ref, bg_ref, w1_ref, b1_ref, w2_ref,
              b2_ref, wg_ref, o_ref):
    dis = _dis_from(degp_ref)
    g = (s_ref[0] + s_ref[1] + hp_ref[...]) * dis + bg_ref[...]
    z = jnp.maximum(_dot(g, w1_ref[...]) + b1_ref[...], 0.0)
    hr = jnp.maximum(_dot(z, w2_ref[...]) + b2_ref[...], 0.0)
    o_ref[...] = _dot(hr, wg_ref[...]) * dis


def _tc3_body(s_ref, hp_ref, degp_ref, bg_ref, w1_ref, b1_ref, w2_ref,
              b2_ref, o_ref):
    dis = _dis_from(degp_ref)
    g = (s_ref[0] + s_ref[1] + hp_ref[...]) * dis + bg_ref[...]
    z = jnp.maximum(_dot(g, w1_ref[...]) + b1_ref[...], 0.0)
    o_ref[...] = _dot(z, w2_ref[...]) + b2_ref[...]


def _full(shape):
    return pl.BlockSpec(shape, lambda i: tuple(0 for _ in shape))


def _tc_mm(x_p, W0):
    return pl.pallas_call(
        _tc_mm_body,
        grid=(GRID,),
        in_specs=[
            pl.BlockSpec((BR, D), lambda i: (i, 0)),
            _full((D, H)),
        ],
        out_specs=pl.BlockSpec((BR, H), lambda i: (i, 0)),
        out_shape=jax.ShapeDtypeStruct((NP, H), f32),
    )(x_p, W0)


def _tc_scale(u, degp):
    return pl.pallas_call(
        _tc_scale_body,
        grid=(GRID,),
        in_specs=[
            pl.BlockSpec((BR, H), lambda i: (i, 0)),
            pl.BlockSpec((2, BR, DEGW), lambda i: (0, i, 0)),
        ],
        out_specs=pl.BlockSpec((BR, H), lambda i: (i, 0)),
        out_shape=jax.ShapeDtypeStruct((NP, H), f32),
    )(u, degp)


def _tc2(s0, h0p, degp, bg, W1, b1, W2, b2, Wg):
    return pl.pallas_call(
        _tc2_body,
        grid=(GRID,),
        in_specs=[
            pl.BlockSpec((2, BR, H), lambda i: (0, i, 0)),
            pl.BlockSpec((BR, H), lambda i: (i, 0)),
            pl.BlockSpec((2, BR, DEGW), lambda i: (0, i, 0)),
            _full((1, H)),
            _full((H, H)),
            _full((1, H)),
            _full((H, HH)),
            _full((1, HH)),
            _full((HH, H)),
        ],
        out_specs=pl.BlockSpec((BR, H), lambda i: (i, 0)),
        out_shape=jax.ShapeDtypeStruct((NP, H), f32),
    )(s0, h0p, degp, bg, W1, b1, W2, b2, Wg)


def _tc3(s1, h1p, degp, bg, W1, b1, W2, b2):
    return pl.pallas_call(
        _tc3_body,
        grid=(GRID,),
        in_specs=[
            pl.BlockSpec((2, BR, H), lambda i: (0, i, 0)),
            pl.BlockSpec((BR, H), lambda i: (i, 0)),
            pl.BlockSpec((2, BR, DEGW), lambda i: (0, i, 0)),
            _full((1, H)),
            _full((H, H)),
            _full((1, H)),
            _full((H, HH)),
            _full((1, HH)),
        ],
        out_specs=pl.BlockSpec((BR, HH), lambda i: (i, 0)),
        out_shape=jax.ShapeDtypeStruct((NP, HH), f32),
    )(s1, h1p, degp, bg, W1, b1, W2, b2)


def kernel(x, edge_index,
           W_gcn0, b_gcn0, W_mlp0_1, b_mlp0_1, W_mlp0_2, b_mlp0_2,
           W_gcn1, b_gcn1, W_mlp1_1, b_mlp1_1, W_mlp1_2, b_mlp1_2):
    row = edge_index[0].astype(i32)
    col = edge_index[1].astype(i32)
    pad = EP - E
    row_p = jnp.concatenate([row, jnp.zeros((pad,), i32)])
    col_p = jnp.concatenate(
        [col, N + (jnp.arange(pad, dtype=i32) % (NP - N))])
    row_p = row_p.reshape(EP // LANES, LANES)
    col_p = col_p.reshape(EP // LANES, LANES)
    x_p = jnp.pad(x, ((0, NP - N), (0, 0)))
    z_deg = jnp.zeros((RPT, DEGW), f32)
    z_h = jnp.zeros((RPT, H), f32)
    ones_deg = jnp.ones((LANES, DEGW), f32)

    degp = _sc_deg(col_p, ones_deg, z_deg)
    u = _tc_mm(x_p, W_gcn0)
    h0p = _tc_scale(u, degp)
    s0 = _sc_prop(h0p, row_p, col_p, z_h)
    h1p = _tc2(s0, h0p, degp, b_gcn0.reshape(1, H),
               W_mlp0_1, b_mlp0_1.reshape(1, H),
               W_mlp0_2, b_mlp0_2.reshape(1, HH), W_gcn1)
    s1 = _sc_prop(h1p, row_p, col_p, z_h)
    out = _tc3(s1, h1p, degp, b_gcn1.reshape(1, H),
               W_mlp1_1, b_mlp1_1.reshape(1, H),
               W_mlp1_2, b_mlp1_2.reshape(1, HH))
    return out[:N]


# speedup vs baseline: 1.2123x; 1.1319x over previous
import functools

import jax
import jax.numpy as jnp
from jax import lax
from jax.experimental import pallas as pl
from jax.experimental.pallas import tpu as pltpu
from jax.experimental.pallas import tpu_sc as plsc

N = 10000
NP = 10240
E = 320000
D = 128
H = 32
HH = H * H
LANES = 128
K = 80
NW = 32
EP = NW * K * LANES
NSUB = 16
RPT = NP // NSUB
TPT = N // NSUB
DEGW = 16
BR = 1000
GRID = N // BR

f32 = jnp.float32
i32 = jnp.int32

_MESH = plsc.VectorSubcoreMesh(core_axis_name="c", subcore_axis_name="s")
_SC_PARAMS = pltpu.CompilerParams(use_tc_tiling_on_sc=False)


def _sc_deg_body(col_hbm, ones_hbm, z_hbm, out_hbm, acc, colv, ones_v):
    c = lax.axis_index("c")
    s = lax.axis_index("s")
    w = c * NSUB + s
    tb = s * RPT
    pltpu.sync_copy(col_hbm.at[pl.ds(w * K, K)], colv)
    pltpu.sync_copy(ones_hbm, ones_v)
    pltpu.sync_copy(z_hbm, acc.at[pl.ds(tb, RPT)])
    plsc.subcore_barrier()

    def step(k, carry):
        pltpu.sync_copy(ones_v, acc.at[colv.at[k]], add=True)
        return carry

    lax.fori_loop(0, K, step, 0)
    plsc.subcore_barrier()
    pltpu.sync_copy(acc.at[pl.ds(tb, RPT)], out_hbm.at[c, pl.ds(tb, RPT)])


_sc_deg = functools.partial(
    pl.kernel,
    out_type=jax.ShapeDtypeStruct((2, NP, DEGW), f32),
    mesh=_MESH,
    scratch_types=[
        pltpu.VMEM_SHARED((NP, DEGW), f32),
        pltpu.VMEM((K, LANES), i32),
        pltpu.VMEM((LANES, DEGW), f32),
    ],
    compiler_params=_SC_PARAMS,
)(_sc_deg_body)


def _sc_prop_body(tab_hbm, row_hbm, col_hbm, z_hbm, out_hbm,
                  acc, tab_sh, rowv, colv, buf0, buf1,
                  sg0, sg1, sg2, sg3):
    c = lax.axis_index("c")
    s = lax.axis_index("s")
    w = c * NSUB + s
    tb = s * RPT
    h1 = pltpu.async_copy(row_hbm.at[pl.ds(w * K, K)], rowv, sg0)
    h2 = pltpu.async_copy(col_hbm.at[pl.ds(w * K, K)], colv, sg1)
    h3 = pltpu.async_copy(z_hbm, acc.at[pl.ds(tb, RPT)], sg2)
    h4 = pltpu.async_copy(tab_hbm.at[pl.ds(s * TPT, TPT)],
                          tab_sh.at[pl.ds(s * TPT, TPT)], sg3)
    h1.wait()
    h2.wait()
    h3.wait()
    h4.wait()
    plsc.subcore_barrier()

    pltpu.async_copy(tab_sh.at[rowv.at[0]], buf0, sg0)

    def pair(k, carry):
        j0 = 2 * k
        j1 = j0 + 1
        pltpu.async_copy(tab_sh.at[rowv.at[j1]], buf1, sg1)
        pltpu.make_async_copy(tab_sh.at[rowv.at[j0]], buf0, sg0).wait()
        pltpu.sync_copy(buf0, acc.at[colv.at[j0]], add=True)

        @pl.when(j1 + 1 < K)
        def _():
            pltpu.async_copy(tab_sh.at[rowv.at[j1 + 1]], buf0, sg0)

        pltpu.make_async_copy(tab_sh.at[rowv.at[j1]], buf1, sg1).wait()
        pltpu.sync_copy(buf1, acc.at[colv.at[j1]], add=True)
        return carry

    lax.fori_loop(0, K // 2, pair, 0)
    plsc.subcore_barrier()
    pltpu.sync_copy(acc.at[pl.ds(tb, RPT)], out_hbm.at[c, pl.ds(tb, RPT)])


_sc_prop = functools.partial(
    pl.kernel,
    out_type=jax.ShapeDtypeStruct((2, NP, H), f32),
    mesh=_MESH,
    scratch_types=[
        pltpu.VMEM_SHARED((NP, H), f32),
        pltpu.VMEM_SHARED((N, H), f32),
        pltpu.VMEM((K, LANES), i32),
        pltpu.VMEM((K, LANES), i32),
        pltpu.VMEM((LANES, H), f32),
        pltpu.VMEM((LANES, H), f32),
        pltpu.SemaphoreType.DMA,
        pltpu.SemaphoreType.DMA,
        pltpu.SemaphoreType.DMA,
        pltpu.SemaphoreType.DMA,
    ],
    compiler_params=_SC_PARAMS,
)(_sc_prop_body)


def _dis_from(degp_ref):
    deg = degp_ref[0, :, 0:1] + degp_ref[1, :, 0:1] + 1.0
    return lax.rsqrt(deg)


def _dot(a, b):
    return jnp.dot(a, b, preferred_element_type=f32,
                   precision=lax.Precision.DEFAULT)


def _tc_mm_body(x_ref, w0_ref, o_ref):
    o_ref[...] = _dot(x_ref[...], w0_ref[...])


def _tc_scale_body(u_ref, degp_ref, o_ref):
    o_ref[...] = u_ref[...] * _dis_from(degp_ref)


def _tc2_body(s_ref, hp_ref, degp_ref, bg_ref, w1_ref, b1_ref, w2_ref,
              b2_ref, wg_ref, o_ref):
    dis = _dis_from(degp_ref)
    g = (s_ref[0] + s_ref[1] + hp_ref[...]) * dis + bg_ref[...]
    z = jnp.maximum(_dot(g, w1_ref[...]) + b1_ref[...], 0.0)
    hr = jnp.maximum(_dot(z, w2_ref[...]) + b2_ref[...], 0.0)
    o_ref[...] = _dot(hr, wg_ref[...]) * dis


def _tc3_body(s_ref, hp_ref, degp_ref, bg_ref, w1_ref, b1_ref, w2_ref,
              b2_ref, o_ref):
    dis = _dis_from(degp_ref)
    g = (s_ref[0] + s_ref[1] + hp_ref[...]) * dis + bg_ref[...]
    z = jnp.maximum(_dot(g, w1_ref[...]) + b1_ref[...], 0.0)
    o_ref[...] = _dot(z, w2_ref[...]) + b2_ref[...]


def _full(shape):
    return pl.BlockSpec(shape, lambda i: tuple(0 for _ in shape))


def _tc_mm(x_p, W0):
    return pl.pallas_call(
        _tc_mm_body,
        grid=(GRID,),
        in_specs=[
            pl.BlockSpec((BR, D), lambda i: (i, 0)),
            _full((D, H)),
        ],
        out_specs=pl.BlockSpec((BR, H), lambda i: (i, 0)),
        out_shape=jax.ShapeDtypeStruct((N, H), f32),
    )(x_p, W0)


def _tc_scale(u, degp):
    return pl.pallas_call(
        _tc_scale_body,
        grid=(GRID,),
        in_specs=[
            pl.BlockSpec((BR, H), lambda i: (i, 0)),
            pl.BlockSpec((2, BR, DEGW), lambda i: (0, i, 0)),
        ],
        out_specs=pl.BlockSpec((BR, H), lambda i: (i, 0)),
        out_shape=jax.ShapeDtypeStruct((N, H), f32),
    )(u, degp)


def _tc2(s0, h0p, degp, bg, W1, b1, W2, b2, Wg):
    return pl.pallas_call(
        _tc2_body,
        grid=(GRID,),
        in_specs=[
            pl.BlockSpec((2, BR, H), lambda i: (0, i, 0)),
            pl.BlockSpec((BR, H), lambda i: (i, 0)),
            pl.BlockSpec((2, BR, DEGW), lambda i: (0, i, 0)),
            _full((1, H)),
            _full((H, H)),
            _full((1, H)),
            _full((H, HH)),
            _full((1, HH)),
            _full((HH, H)),
        ],
        out_specs=pl.BlockSpec((BR, H), lambda i: (i, 0)),
        out_shape=jax.ShapeDtypeStruct((N, H), f32),
    )(s0, h0p, degp, bg, W1, b1, W2, b2, Wg)


def _tc3(s1, h1p, degp, bg, W1, b1, W2, b2):
    return pl.pallas_call(
        _tc3_body,
        grid=(GRID,),
        in_specs=[
            pl.BlockSpec((2, BR, H), lambda i: (0, i, 0)),
            pl.BlockSpec((BR, H), lambda i: (i, 0)),
            pl.BlockSpec((2, BR, DEGW), lambda i: (0, i, 0)),
            _full((1, H)),
            _full((H, H)),
            _full((1, H)),
            _full((H, HH)),
            _full((1, HH)),
        ],
        out_specs=pl.BlockSpec((BR, HH), lambda i: (i, 0)),
        out_shape=jax.ShapeDtypeStruct((N, HH), f32),
    )(s1, h1p, degp, bg, W1, b1, W2, b2)


def kernel(x, edge_index,
           W_gcn0, b_gcn0, W_mlp0_1, b_mlp0_1, W_mlp0_2, b_mlp0_2,
           W_gcn1, b_gcn1, W_mlp1_1, b_mlp1_1, W_mlp1_2, b_mlp1_2):
    row = edge_index[0].astype(i32)
    col = edge_index[1].astype(i32)
    pad = EP - E
    row_p = jnp.concatenate([row, jnp.zeros((pad,), i32)])
    col_p = jnp.concatenate(
        [col, N + (jnp.arange(pad, dtype=i32) % (NP - N))])
    row_p = row_p.reshape(EP // LANES, LANES)
    col_p = col_p.reshape(EP // LANES, LANES)
    z_deg = jnp.zeros((RPT, DEGW), f32)
    z_h = jnp.zeros((RPT, H), f32)
    ones_deg = jnp.ones((LANES, DEGW), f32)

    degp = _sc_deg(col_p, ones_deg, z_deg)
    u = _tc_mm(x, W_gcn0)
    h0p = _tc_scale(u, degp)
    s0 = _sc_prop(h0p, row_p, col_p, z_h)
    h1p = _tc2(s0, h0p, degp, b_gcn0.reshape(1, H),
               W_mlp0_1, b_mlp0_1.reshape(1, H),
               W_mlp0_2, b_mlp0_2.reshape(1, HH), W_gcn1)
    s1 = _sc_prop(h1p, row_p, col_p, z_h)
    return _tc3(s1, h1p, degp, b_gcn1.reshape(1, H),
                W_mlp1_1, b_mlp1_1.reshape(1, H),
                W_mlp1_2, b_mlp1_2.reshape(1, HH))


# speedup vs baseline: 1.2200x; 1.0064x over previous
import functools

import jax
import jax.numpy as jnp
from jax import lax
from jax.experimental import pallas as pl
from jax.experimental.pallas import tpu as pltpu
from jax.experimental.pallas import tpu_sc as plsc

N = 10000
NP = 10240
E = 320000
D = 128
H = 32
HH = H * H
LANES = 128
K = 80
NW = 32
EP = NW * K * LANES
NSUB = 16
RPT = NP // NSUB
TPT = N // NSUB
DEGW = 8
BR = 1000
GRID = N // BR

f32 = jnp.float32
i32 = jnp.int32

_MESH = plsc.VectorSubcoreMesh(core_axis_name="c", subcore_axis_name="s")
_SC_PARAMS = pltpu.CompilerParams(use_tc_tiling_on_sc=False)


def _sc_deg_body(col_hbm, ones_hbm, z_hbm, out_hbm, acc, colv, ones_v):
    c = lax.axis_index("c")
    s = lax.axis_index("s")
    w = c * NSUB + s
    tb = s * RPT
    pltpu.sync_copy(col_hbm.at[pl.ds(w * K, K)], colv)
    pltpu.sync_copy(ones_hbm, ones_v)
    pltpu.sync_copy(z_hbm, acc.at[pl.ds(tb, RPT)])
    plsc.subcore_barrier()

    def step(k, carry):
        pltpu.sync_copy(ones_v, acc.at[colv.at[k]], add=True)
        return carry

    lax.fori_loop(0, K, step, 0)
    plsc.subcore_barrier()
    pltpu.sync_copy(acc.at[pl.ds(tb, RPT)], out_hbm.at[c, pl.ds(tb, RPT)])


_sc_deg = functools.partial(
    pl.kernel,
    out_type=jax.ShapeDtypeStruct((2, NP, DEGW), f32),
    mesh=_MESH,
    scratch_types=[
        pltpu.VMEM_SHARED((NP, DEGW), f32),
        pltpu.VMEM((K, LANES), i32),
        pltpu.VMEM((LANES, DEGW), f32),
    ],
    compiler_params=_SC_PARAMS,
)(_sc_deg_body)


def _sc_prop_body(tab_hbm, row_hbm, col_hbm, z_hbm, out_hbm,
                  acc, tab_sh, rowv, colv, buf0, buf1,
                  sg0, sg1, sg2, sg3):
    c = lax.axis_index("c")
    s = lax.axis_index("s")
    w = c * NSUB + s
    tb = s * RPT
    h1 = pltpu.async_copy(row_hbm.at[pl.ds(w * K, K)], rowv, sg0)
    h2 = pltpu.async_copy(col_hbm.at[pl.ds(w * K, K)], colv, sg1)
    h3 = pltpu.async_copy(z_hbm, acc.at[pl.ds(tb, RPT)], sg2)
    h4 = pltpu.async_copy(tab_hbm.at[pl.ds(s * TPT, TPT)],
                          tab_sh.at[pl.ds(s * TPT, TPT)], sg3)
    h1.wait()
    h2.wait()
    h3.wait()
    h4.wait()
    plsc.subcore_barrier()

    pltpu.async_copy(tab_sh.at[rowv.at[0]], buf0, sg0)

    def pair(k, carry):
        j0 = 2 * k
        j1 = j0 + 1
        pltpu.async_copy(tab_sh.at[rowv.at[j1]], buf1, sg1)
        pltpu.make_async_copy(tab_sh.at[rowv.at[j0]], buf0, sg0).wait()
        pltpu.sync_copy(buf0, acc.at[colv.at[j0]], add=True)

        @pl.when(j1 + 1 < K)
        def _():
            pltpu.async_copy(tab_sh.at[rowv.at[j1 + 1]], buf0, sg0)

        pltpu.make_async_copy(tab_sh.at[rowv.at[j1]], buf1, sg1).wait()
        pltpu.sync_copy(buf1, acc.at[colv.at[j1]], add=True)
        return carry

    lax.fori_loop(0, K // 2, pair, 0)
    plsc.subcore_barrier()
    pltpu.sync_copy(acc.at[pl.ds(tb, RPT)], out_hbm.at[c, pl.ds(tb, RPT)])


_sc_prop = functools.partial(
    pl.kernel,
    out_type=jax.ShapeDtypeStruct((2, NP, H), f32),
    mesh=_MESH,
    scratch_types=[
        pltpu.VMEM_SHARED((NP, H), f32),
        pltpu.VMEM_SHARED((N, H), f32),
        pltpu.VMEM((K, LANES), i32),
        pltpu.VMEM((K, LANES), i32),
        pltpu.VMEM((LANES, H), f32),
        pltpu.VMEM((LANES, H), f32),
        pltpu.SemaphoreType.DMA,
        pltpu.SemaphoreType.DMA,
        pltpu.SemaphoreType.DMA,
        pltpu.SemaphoreType.DMA,
    ],
    compiler_params=_SC_PARAMS,
)(_sc_prop_body)


def _dis_from(degp_ref):
    deg = degp_ref[0, :, 0:1] + degp_ref[1, :, 0:1] + 1.0
    return lax.rsqrt(deg)


def _dot(a, b):
    return jnp.dot(a, b, preferred_element_type=f32,
                   precision=lax.Precision.DEFAULT)


def _tc_mm_body(x_ref, w0_ref, o_ref):
    o_ref[...] = _dot(x_ref[...], w0_ref[...])


def _tc_scale_body(u_ref, degp_ref, o_ref):
    o_ref[...] = u_ref[...] * _dis_from(degp_ref)


def _tc2_body(s_ref, hp_ref, degp_ref, bg_ref, w1_ref, b1_ref, w2_ref,
              b2_ref, wg_ref, o_ref):
    dis = _dis_from(degp_ref)
    g = (s_ref[0] + s_ref[1] + hp_ref[...]) * dis + bg_ref[...]
    z = jnp.maximum(_dot(g, w1_ref[...]) + b1_ref[...], 0.0)
    hr = jnp.maximum(_dot(z, w2_ref[...]) + b2_ref[...], 0.0)
    o_ref[...] = _dot(hr, wg_ref[...]) * dis


def _tc3_body(s_ref, hp_ref, degp_ref, bg_ref, w1_ref, b1_ref, w2_ref,
              b2_ref, o_ref):
    dis = _dis_from(degp_ref)
    g = (s_ref[0] + s_ref[1] + hp_ref[...]) * dis + bg_ref[...]
    z = jnp.maximum(_dot(g, w1_ref[...]) + b1_ref[...], 0.0)
    o_ref[...] = _dot(z, w2_ref[...]) + b2_ref[...]


def _full(shape):
    return pl.BlockSpec(shape, lambda i: tuple(0 for _ in shape))


def _tc_mm(x_p, W0):
    return pl.pallas_call(
        _tc_mm_body,
        grid=(GRID,),
        in_specs=[
            pl.BlockSpec((BR, D), lambda i: (i, 0)),
            _full((D, H)),
        ],
        out_specs=pl.BlockSpec((BR, H), lambda i: (i, 0)),
        out_shape=jax.ShapeDtypeStruct((N, H), f32),
    )(x_p, W0)


def _tc_scale(u, degp):
    return pl.pallas_call(
        _tc_scale_body,
        grid=(GRID,),
        in_specs=[
            pl.BlockSpec((BR, H), lambda i: (i, 0)),
            pl.BlockSpec((2, BR, DEGW), lambda i: (0, i, 0)),
        ],
        out_specs=pl.BlockSpec((BR, H), lambda i: (i, 0)),
        out_shape=jax.ShapeDtypeStruct((N, H), f32),
    )(u, degp)


def _tc2(s0, h0p, degp, bg, W1, b1, W2, b2, Wg):
    return pl.pallas_call(
        _tc2_body,
        grid=(GRID,),
        in_specs=[
            pl.BlockSpec((2, BR, H), lambda i: (0, i, 0)),
            pl.BlockSpec((BR, H), lambda i: (i, 0)),
            pl.BlockSpec((2, BR, DEGW), lambda i: (0, i, 0)),
            _full((1, H)),
            _full((H, H)),
            _full((1, H)),
            _full((H, HH)),
            _full((1, HH)),
            _full((HH, H)),
        ],
        out_specs=pl.BlockSpec((BR, H), lambda i: (i, 0)),
        out_shape=jax.ShapeDtypeStruct((N, H), f32),
    )(s0, h0p, degp, bg, W1, b1, W2, b2, Wg)


def _tc3(s1, h1p, degp, bg, W1, b1, W2, b2):
    return pl.pallas_call(
        _tc3_body,
        grid=(GRID,),
        in_specs=[
            pl.BlockSpec((2, BR, H), lambda i: (0, i, 0)),
            pl.BlockSpec((BR, H), lambda i: (i, 0)),
            pl.BlockSpec((2, BR, DEGW), lambda i: (0, i, 0)),
            _full((1, H)),
            _full((H, H)),
            _full((1, H)),
            _full((H, HH)),
            _full((1, HH)),
        ],
        out_specs=pl.BlockSpec((BR, HH), lambda i: (i, 0)),
        out_shape=jax.ShapeDtypeStruct((N, HH), f32),
    )(s1, h1p, degp, bg, W1, b1, W2, b2)


def kernel(x, edge_index,
           W_gcn0, b_gcn0, W_mlp0_1, b_mlp0_1, W_mlp0_2, b_mlp0_2,
           W_gcn1, b_gcn1, W_mlp1_1, b_mlp1_1, W_mlp1_2, b_mlp1_2):
    row = edge_index[0].astype(i32)
    col = edge_index[1].astype(i32)
    pad = EP - E
    row_p = jnp.concatenate([row, jnp.zeros((pad,), i32)])
    col_p = jnp.concatenate(
        [col, N + (jnp.arange(pad, dtype=i32) % (NP - N))])
    row_p = row_p.reshape(EP // LANES, LANES)
    col_p = col_p.reshape(EP // LANES, LANES)
    z_deg = jnp.zeros((RPT, DEGW), f32)
    z_h = jnp.zeros((RPT, H), f32)
    ones_deg = jnp.ones((LANES, DEGW), f32)

    degp = _sc_deg(col_p, ones_deg, z_deg)
    u = _tc_mm(x, W_gcn0)
    h0p = _tc_scale(u, degp)
    s0 = _sc_prop(h0p, row_p, col_p, z_h)
    h1p = _tc2(s0, h0p, degp, b_gcn0.reshape(1, H),
               W_mlp0_1, b_mlp0_1.reshape(1, H),
               W_mlp0_2, b_mlp0_2.reshape(1, HH), W_gcn1)
    s1 = _sc_prop(h1p, row_p, col_p, z_h)
    return _tc3(s1, h1p, degp, b_gcn1.reshape(1, H),
                W_mlp1_1, b_mlp1_1.reshape(1, H),
                W_mlp1_2, b_mlp1_2.reshape(1, HH))


# speedup vs baseline: 1.2260x; 1.0049x over previous
import functools

import jax
import jax.numpy as jnp
from jax import lax
from jax.experimental import pallas as pl
from jax.experimental.pallas import tpu as pltpu
from jax.experimental.pallas import tpu_sc as plsc

N = 10000
NP = 10240
E = 320000
D = 128
H = 32
HH = H * H
LANES = 128
K = 80
NW = 32
EP = NW * K * LANES
NSUB = 16
RPT = NP // NSUB
TPT = N // NSUB
DEGW = 8
BR = 1000
GRID = N // BR

f32 = jnp.float32
i32 = jnp.int32

_MESH = plsc.VectorSubcoreMesh(core_axis_name="c", subcore_axis_name="s")
_SC_PARAMS = pltpu.CompilerParams(use_tc_tiling_on_sc=False)


def _sc_deg_body(col_hbm, ones_hbm, z_hbm, out_hbm, acc, colv, ones_v):
    c = lax.axis_index("c")
    s = lax.axis_index("s")
    w = c * NSUB + s
    tb = s * RPT
    pltpu.sync_copy(col_hbm.at[pl.ds(w * K, K)], colv)
    pltpu.sync_copy(ones_hbm, ones_v)
    pltpu.sync_copy(z_hbm, acc.at[pl.ds(tb, RPT)])
    plsc.subcore_barrier()

    def step(k, carry):
        pltpu.sync_copy(ones_v, acc.at[colv.at[k]], add=True)
        return carry

    lax.fori_loop(0, K, step, 0)
    plsc.subcore_barrier()
    pltpu.sync_copy(acc.at[pl.ds(tb, RPT)], out_hbm.at[c, pl.ds(tb, RPT)])


_sc_deg = functools.partial(
    pl.kernel,
    out_type=jax.ShapeDtypeStruct((2, NP, DEGW), f32),
    mesh=_MESH,
    scratch_types=[
        pltpu.VMEM_SHARED((NP, DEGW), f32),
        pltpu.VMEM((K, LANES), i32),
        pltpu.VMEM((LANES, DEGW), f32),
    ],
    compiler_params=_SC_PARAMS,
)(_sc_deg_body)


def _sc_prop_body(tab_hbm, row_hbm, col_hbm, z_hbm, out_hbm,
                  acc, tab_sh, rowv, colv, buf0, buf1,
                  sg0, sg1, sg2, sg3):
    c = lax.axis_index("c")
    s = lax.axis_index("s")
    w = c * NSUB + s
    tb = s * RPT
    h1 = pltpu.async_copy(row_hbm.at[pl.ds(w * K, K)], rowv, sg0)
    h2 = pltpu.async_copy(col_hbm.at[pl.ds(w * K, K)], colv, sg1)
    h3 = pltpu.async_copy(z_hbm, acc.at[pl.ds(tb, RPT)], sg2)
    h4 = pltpu.async_copy(tab_hbm.at[pl.ds(s * TPT, TPT)],
                          tab_sh.at[pl.ds(s * TPT, TPT)], sg3)
    h1.wait()
    h2.wait()
    h3.wait()
    h4.wait()
    plsc.subcore_barrier()

    pltpu.async_copy(tab_sh.at[rowv.at[0]], buf0, sg0)

    def pair(k, carry):
        j0 = 2 * k
        j1 = j0 + 1
        pltpu.async_copy(tab_sh.at[rowv.at[j1]], buf1, sg1)
        pltpu.make_async_copy(tab_sh.at[rowv.at[j0]], buf0, sg0).wait()
        pltpu.sync_copy(buf0, acc.at[colv.at[j0]], add=True)

        @pl.when(j1 + 1 < K)
        def _():
            pltpu.async_copy(tab_sh.at[rowv.at[j1 + 1]], buf0, sg0)

        pltpu.make_async_copy(tab_sh.at[rowv.at[j1]], buf1, sg1).wait()
        pltpu.sync_copy(buf1, acc.at[colv.at[j1]], add=True)
        return carry

    lax.fori_loop(0, K // 2, pair, 0)
    plsc.subcore_barrier()
    pltpu.sync_copy(acc.at[pl.ds(tb, RPT)], out_hbm.at[c, pl.ds(tb, RPT)])


_sc_prop = functools.partial(
    pl.kernel,
    out_type=jax.ShapeDtypeStruct((2, NP, H), f32),
    mesh=_MESH,
    scratch_types=[
        pltpu.VMEM_SHARED((NP, H), f32),
        pltpu.VMEM_SHARED((N, H), f32),
        pltpu.VMEM((K, LANES), i32),
        pltpu.VMEM((K, LANES), i32),
        pltpu.VMEM((LANES, H), f32),
        pltpu.VMEM((LANES, H), f32),
        pltpu.SemaphoreType.DMA,
        pltpu.SemaphoreType.DMA,
        pltpu.SemaphoreType.DMA,
        pltpu.SemaphoreType.DMA,
    ],
    compiler_params=_SC_PARAMS,
)(_sc_prop_body)


def _dis_from(degp_ref):
    deg = degp_ref[0, :, 0:1] + degp_ref[1, :, 0:1] + 1.0
    return lax.rsqrt(deg)


def _dot(a, b):
    return jnp.dot(a, b, preferred_element_type=f32,
                   precision=lax.Precision.DEFAULT)


def _tc1_body(x_ref, degp_ref, w0_ref, o_ref):
    o_ref[...] = _dot(x_ref[...], w0_ref[...]) * _dis_from(degp_ref)


def _tc2_body(s_ref, hp_ref, degp_ref, bg_ref, w1_ref, b1_ref, w2_ref,
              b2_ref, wg_ref, o_ref):
    dis = _dis_from(degp_ref)
    g = (s_ref[0] + s_ref[1] + hp_ref[...]) * dis + bg_ref[...]
    z = jnp.maximum(_dot(g, w1_ref[...]) + b1_ref[...], 0.0)
    hr = jnp.maximum(_dot(z, w2_ref[...]) + b2_ref[...], 0.0)
    o_ref[...] = _dot(hr, wg_ref[...]) * dis


def _tc3_body(s_ref, hp_ref, degp_ref, bg_ref, w1_ref, b1_ref, w2_ref,
              b2_ref, o_ref):
    dis = _dis_from(degp_ref)
    g = (s_ref[0] + s_ref[1] + hp_ref[...]) * dis + bg_ref[...]
    z = jnp.maximum(_dot(g, w1_ref[...]) + b1_ref[...], 0.0)
    o_ref[...] = _dot(z, w2_ref[...]) + b2_ref[...]


def _full(shape):
    return pl.BlockSpec(shape, lambda i: tuple(0 for _ in shape))


def _tc1(x, degp, W0):
    return pl.pallas_call(
        _tc1_body,
        grid=(GRID,),
        in_specs=[
            pl.BlockSpec((BR, D), lambda i: (i, 0)),
            pl.BlockSpec((2, BR, DEGW), lambda i: (0, i, 0)),
            _full((D, H)),
        ],
        out_specs=pl.BlockSpec((BR, H), lambda i: (i, 0)),
        out_shape=jax.ShapeDtypeStruct((N, H), f32),
    )(x, degp, W0)


def _tc2(s0, h0p, degp, bg, W1, b1, W2, b2, Wg):
    return pl.pallas_call(
        _tc2_body,
        grid=(GRID,),
        in_specs=[
            pl.BlockSpec((2, BR, H), lambda i: (0, i, 0)),
            pl.BlockSpec((BR, H), lambda i: (i, 0)),
            pl.BlockSpec((2, BR, DEGW), lambda i: (0, i, 0)),
            _full((1, H)),
            _full((H, H)),
            _full((1, H)),
            _full((H, HH)),
            _full((1, HH)),
            _full((HH, H)),
        ],
        out_specs=pl.BlockSpec((BR, H), lambda i: (i, 0)),
        out_shape=jax.ShapeDtypeStruct((N, H), f32),
    )(s0, h0p, degp, bg, W1, b1, W2, b2, Wg)


def _tc3(s1, h1p, degp, bg, W1, b1, W2, b2):
    return pl.pallas_call(
        _tc3_body,
        grid=(GRID,),
        in_specs=[
            pl.BlockSpec((2, BR, H), lambda i: (0, i, 0)),
            pl.BlockSpec((BR, H), lambda i: (i, 0)),
            pl.BlockSpec((2, BR, DEGW), lambda i: (0, i, 0)),
            _full((1, H)),
            _full((H, H)),
            _full((1, H)),
            _full((H, HH)),
            _full((1, HH)),
        ],
        out_specs=pl.BlockSpec((BR, HH), lambda i: (i, 0)),
        out_shape=jax.ShapeDtypeStruct((N, HH), f32),
    )(s1, h1p, degp, bg, W1, b1, W2, b2)


def kernel(x, edge_index,
           W_gcn0, b_gcn0, W_mlp0_1, b_mlp0_1, W_mlp0_2, b_mlp0_2,
           W_gcn1, b_gcn1, W_mlp1_1, b_mlp1_1, W_mlp1_2, b_mlp1_2):
    row = edge_index[0].astype(i32)
    col = edge_index[1].astype(i32)
    pad = EP - E
    row_p = jnp.concatenate([row, jnp.zeros((pad,), i32)])
    col_p = jnp.concatenate(
        [col, N + (jnp.arange(pad, dtype=i32) % (NP - N))])
    row_p = row_p.reshape(EP // LANES, LANES)
    col_p = col_p.reshape(EP // LANES, LANES)
    z_deg = jnp.zeros((RPT, DEGW), f32)
    z_h = jnp.zeros((RPT, H), f32)
    ones_deg = jnp.ones((LANES, DEGW), f32)

    degp = _sc_deg(col_p, ones_deg, z_deg)
    h0p = _tc1(x, degp, W_gcn0)
    s0 = _sc_prop(h0p, row_p, col_p, z_h)
    h1p = _tc2(s0, h0p, degp, b_gcn0.reshape(1, H),
               W_mlp0_1, b_mlp0_1.reshape(1, H),
               W_mlp0_2, b_mlp0_2.reshape(1, HH), W_gcn1)
    s1 = _sc_prop(h1p, row_p, col_p, z_h)
    return _tc3(s1, h1p, degp, b_gcn1.reshape(1, H),
                W_mlp1_1, b_mlp1_1.reshape(1, H),
                W_mlp1_2, b_mlp1_2.reshape(1, HH))
